# baseline (speedup 1.0000x reference)
import numpy as np

import jax
import jax.numpy as jnp
from jax.experimental import pallas as pl
from jax.experimental.pallas import tpu as pltpu

_ROWS_PER_BLOCK = 512


def _build_masks():
    def qr(npad, pairs):
        q = np.zeros((npad, 128), np.float32)
        r = np.zeros((npad, 128), np.float32)
        for lane, row, col in pairs:
            q[col, lane] = 1.0
            r[row, lane] = 1.0
        return q, r

    qc, rc = qr(8, [(l, l // 7, l % 7) for l in range(49)])
    qs, rs = qr(8, [(49 + j, j // 8, j % 8) for j in range(64)])
    qg, rg = qr(8, [(113, 0, 0), (114, 0, 1), (115, 1, 1)])
    sym = [(lo, hi) for lo in range(17) for hi in range(lo, 17)]
    qm0, rm0 = qr(24, [(k, lo, hi) for k, (lo, hi) in enumerate(sym[:128])])
    qm1, rm1 = qr(24, [(k - 128, lo, hi)
                       for k, (lo, hi) in enumerate(sym) if k >= 128])
    return np.concatenate([qc, rc, qs, rs, qg, rg, qm0, rm0, qm1, rm1], 0)


_MASKS = _build_masks()


def _mm(a, b, ca, cb):
    return jax.lax.dot_general(a, b, (((ca,), (cb,)), ((), ())),
                               preferred_element_type=jnp.float32)


def _build_body(ge_ref, ce_ref, se_ref, me_ref, fw_ref, fb_ref, wt_ref,
                mk_ref, ta_ref, tm_ref):
    bias = fb_ref[0]
    w = [wt_ref[i] * fw_ref[0, i] for i in range(4)]

    def flat(dots, ncol, q, r, wf):
        g = _mm(dots, q, 1, 0)
        dflat = jnp.sum(g * r, axis=0, keepdims=True)
        nr = _mm(ncol, r, 0, 0)
        nc = _mm(ncol, q, 0, 0)
        return dflat * wf / jnp.maximum(nr * nc, 1e-8)

    def prep(e):
        ev = e[...]
        dots = _mm(ev, ev, 1, 1)
        ncol = jnp.sqrt(jnp.sum(ev * ev, axis=1, keepdims=True))
        return dots, ncol

    cd, cn = prep(ce_ref)
    fc = flat(cd, cn, mk_ref[0:7, :], mk_ref[8:15, :], w[1])
    sd, sn = prep(se_ref)
    fs = flat(sd, sn, mk_ref[16:24, :], mk_ref[24:32, :], w[2])
    gd, gn = prep(ge_ref)
    fg = flat(gd, gn, mk_ref[32:34, :], mk_ref[40:42, :], w[0])
    gvalid = jnp.sum(mk_ref[40:42, :], axis=0, keepdims=True)
    ta_row = fc + fs + fg + bias * gvalid
    ta_ref[...] = pltpu.repeat(ta_row, 8, axis=0)

    md, mn = prep(me_ref)
    fm0 = flat(md, mn, mk_ref[48:65, :], mk_ref[72:89, :], w[3])
    fm1 = flat(md, mn, mk_ref[96:113, :], mk_ref[120:137, :], w[3])

    def rtne16(x):
        u = pltpu.bitcast(x, jnp.uint32)
        return (u + 0x7FFF + ((u >> 16) & 1)) >> 16

    lo16 = rtne16(fm0)
    hi16 = rtne16(fm1) << 16
    tm_row = pltpu.bitcast(lo16 | hi16, jnp.int32)
    tm_ref[...] = pltpu.repeat(tm_row, 8, axis=0)


def _build_tables(gender_emb, college_emb, school_emb, mbti_emb,
                  final_fc_w, final_fc_b, weight):
    f32 = jnp.float32
    vm = pl.BlockSpec(memory_space=pltpu.VMEM)
    sm = pl.BlockSpec(memory_space=pltpu.SMEM)
    return pl.pallas_call(
        _build_body,
        out_shape=(jax.ShapeDtypeStruct((8, 128), jnp.float32),
                   jax.ShapeDtypeStruct((8, 128), jnp.int32)),
        in_specs=[vm, vm, vm, vm, sm, sm, sm, vm],
        out_specs=(vm, vm),
    )(gender_emb.astype(f32), college_emb.astype(f32),
      school_emb.astype(f32), mbti_emb.astype(f32),
      final_fc_w.astype(f32), final_fc_b.astype(f32), weight.astype(f32),
      jnp.asarray(_MASKS))


def _lookup_body(ta_ref, tm_ref, ag_ref, asc_ref, aco_ref, am_ref,
                 bg_ref, bsc_ref, bco_ref, bm_ref, out_ref):
    rb = out_ref.shape[0]
    rep = rb // 8
    ta8 = ta_ref[0:8, :]
    ta = pltpu.repeat(ta8, rep, axis=0)
    tm = pltpu.repeat(tm_ref[0:8, :], rep, axis=0)
    t00 = ta8[0, 113]
    t01 = ta8[0, 114]
    t11 = ta8[0, 115]

    kc = aco_ref[...] * 7 + bco_ref[...]
    ks = asc_ref[...] * 8 + bsc_ref[...] + 49
    am = am_ref[...]
    bm = bm_ref[...]
    lo = jnp.minimum(am, bm)
    hi = jnp.maximum(am, bm)
    ksym = lo * 17 - ((lo * lo + lo) >> 1) + hi
    kw = ksym & 127

    take = lambda t, i: jnp.take_along_axis(t, i, axis=1,
                                            mode="promise_in_bounds")
    vc = take(ta, kc)
    vs = take(ta, ks)
    gm = take(tm, kw)
    mbits = jnp.where(ksym < 128, gm << 16, gm & jnp.int32(-65536))
    vm = pltpu.bitcast(mbits, jnp.float32)

    kg = ag_ref[...] + bg_ref[...]
    vg = jnp.where(kg == 0, t00, jnp.where(kg == 1, t01, t11))

    out_ref[...] = jax.nn.sigmoid(vg + vc + vs + vm)


def kernel(gender_emb, college_emb, school_emb, mbti_emb, final_fc_w,
           final_fc_b, weight,
           userA_gender, userA_school, userA_college, userA_mbti,
           userB_gender, userB_school, userB_college, userB_mbti):
    ta8, tm8 = _build_tables(gender_emb, college_emb, school_emb, mbti_emb,
                             final_fc_w, final_fc_b, weight)

    B = userA_gender.shape[0]
    assert B % (1024 * 1024) == 0
    M, N = B // 1024, 1024

    def to2d(x):
        return x.astype(jnp.int32).reshape(M, N)

    idxs = [to2d(x) for x in (userA_gender, userA_school, userA_college,
                              userA_mbti, userB_gender, userB_school,
                              userB_college, userB_mbti)]

    blk = pl.BlockSpec((512, 128), lambda r, c: (r, c))
    tblk = pl.BlockSpec((8, 128), lambda r, c: (0, 0))
    out = pl.pallas_call(
        _lookup_body,
        out_shape=jax.ShapeDtypeStruct((M, N), jnp.float32),
        grid=(2, 8),
        in_specs=[tblk, tblk] + [blk] * 8,
        out_specs=blk,
        compiler_params=pltpu.CompilerParams(
            dimension_semantics=("parallel", "arbitrary"),
            vmem_limit_bytes=32 << 20,
        ),
    )(ta8, tm8, *idxs)

    return out.reshape(-1)[:B].reshape(B, 1)


# speedup vs baseline: 2.9593x; 2.9593x over previous
import numpy as np

import jax
import jax.numpy as jnp
from jax.experimental import pallas as pl
from jax.experimental.pallas import tpu as pltpu

_ROWS_PER_BLOCK = 2048


def _build_masks():
    def qr(npad, pairs):
        q = np.zeros((npad, 128), np.float32)
        r = np.zeros((npad, 128), np.float32)
        for lane, row, col in pairs:
            q[col, lane] = 1.0
            r[row, lane] = 1.0
        return q, r

    qc, rc = qr(8, [(l, l // 7, l % 7) for l in range(49)])
    qs, rs = qr(8, [(49 + j, j // 8, j % 8) for j in range(64)])
    qg, rg = qr(8, [(113, 0, 0), (114, 0, 1), (115, 1, 1)])
    sym = [(lo, hi) for lo in range(17) for hi in range(lo, 17)]
    qm0, rm0 = qr(24, [(k, lo, hi) for k, (lo, hi) in enumerate(sym[:128])])
    qm1, rm1 = qr(24, [(k - 128, lo, hi)
                       for k, (lo, hi) in enumerate(sym) if k >= 128])
    return np.concatenate([qc, rc, qs, rs, qg, rg, qm0, rm0, qm1, rm1], 0)


_MASKS = _build_masks()


def _mm(a, b, ca, cb):
    return jax.lax.dot_general(a, b, (((ca,), (cb,)), ((), ())),
                               preferred_element_type=jnp.float32)


def _build_tables(ge_ref, ce_ref, se_ref, me_ref, fw_ref, fb_ref, wt_ref,
                  mk_ref, ta_ref, tm_ref):
    bias = fb_ref[0]
    w = [wt_ref[i] * fw_ref[0, i] for i in range(4)]

    def flat(dots, ncol, q, r, wf):
        g = _mm(dots, q, 1, 0)
        dflat = jnp.sum(g * r, axis=0, keepdims=True)
        nr = _mm(ncol, r, 0, 0)
        nc = _mm(ncol, q, 0, 0)
        return dflat * wf / jnp.maximum(nr * nc, 1e-8)

    def prep(e):
        ev = e[...]
        dots = _mm(ev, ev, 1, 1)
        ncol = jnp.sqrt(jnp.sum(ev * ev, axis=1, keepdims=True))
        return dots, ncol

    cd, cn = prep(ce_ref)
    fc = flat(cd, cn, mk_ref[0:7, :], mk_ref[8:15, :], w[1])
    sd, sn = prep(se_ref)
    fs = flat(sd, sn, mk_ref[16:24, :], mk_ref[24:32, :], w[2])
    gd, gn = prep(ge_ref)
    fg = flat(gd, gn, mk_ref[32:34, :], mk_ref[40:42, :], w[0])
    gvalid = jnp.sum(mk_ref[40:42, :], axis=0, keepdims=True)
    ta_row = fc + fs + fg + bias * gvalid
    ta_ref[...] = pltpu.repeat(ta_row, 8, axis=0)

    md, mn = prep(me_ref)
    fm0 = flat(md, mn, mk_ref[48:65, :], mk_ref[72:89, :], w[3])
    fm1 = flat(md, mn, mk_ref[96:113, :], mk_ref[120:137, :], w[3])

    def rtne16(x):
        u = pltpu.bitcast(x, jnp.uint32)
        return (u + 0x7FFF + ((u >> 16) & 1)) >> 16

    lo16 = rtne16(fm0)
    hi16 = rtne16(fm1) << 16
    tm_row = pltpu.bitcast(lo16 | hi16, jnp.int32)
    tm_ref[...] = pltpu.repeat(tm_row, 8, axis=0)


def _fused_body(ge_ref, ce_ref, se_ref, me_ref, fw_ref, fb_ref, wt_ref,
                mk_ref, pk_ref, out_ref, ta_s, tm_s):
    @pl.when(pl.program_id(0) == 0)
    def _():
        _build_tables(ge_ref, ce_ref, se_ref, me_ref, fw_ref, fb_ref,
                      wt_ref, mk_ref, ta_s, tm_s)

    rb = out_ref.shape[0]
    rep = rb // 8
    ta8 = ta_s[0:8, :]
    ta = pltpu.repeat(ta8, rep, axis=0)
    tm = pltpu.repeat(tm_s[0:8, :], rep, axis=0)
    t00 = ta8[0, 113]
    t01 = ta8[0, 114]
    t11 = ta8[0, 115]

    p = pk_ref[...]
    kg = p & 3
    kc = (p >> 2) & 63
    ks = (p >> 8) & 127
    am = (p >> 15) & 31
    bm = p >> 20
    lo = jnp.minimum(am, bm)
    hi = jnp.maximum(am, bm)
    ksym = lo * 17 - ((lo * lo + lo) >> 1) + hi
    kw = ksym & 127

    take = lambda t, i: jnp.take_along_axis(t, i, axis=1,
                                            mode="promise_in_bounds")
    vc = take(ta, kc)
    vs = take(ta, ks)
    gm = take(tm, kw)
    mbits = jnp.where(ksym < 128, gm << 16, gm & jnp.int32(-65536))
    vm = pltpu.bitcast(mbits, jnp.float32)

    vg = jnp.where(kg == 0, t00, jnp.where(kg == 1, t01, t11))

    out_ref[...] = jax.nn.sigmoid(vg + vc + vs + vm)


def kernel(gender_emb, college_emb, school_emb, mbti_emb, final_fc_w,
           final_fc_b, weight,
           userA_gender, userA_school, userA_college, userA_mbti,
           userB_gender, userB_school, userB_college, userB_mbti):
    B = userA_gender.shape[0]
    rb = _ROWS_PER_BLOCK
    rows = pl.cdiv(B, 128)
    rows = pl.cdiv(rows, rb) * rb
    bpad = rows * 128 - B

    ag, asc, aco, am, bg, bsc, bco, bm = (
        x.astype(jnp.int32) for x in (userA_gender, userA_school,
                                      userA_college, userA_mbti, userB_gender,
                                      userB_school, userB_college, userB_mbti))
    packed = ((ag + bg)
              | ((aco * 7 + bco) << 2)
              | ((asc * 8 + bsc + 49) << 8)
              | (am << 15)
              | (bm << 20))
    if bpad:
        packed = jnp.pad(packed, (0, bpad))
    packed = packed.reshape(rows, 128)

    f32 = jnp.float32
    grid = rows // rb
    blk = pl.BlockSpec((rb, 128), lambda i: (i, 0))
    vm = pl.BlockSpec(memory_space=pltpu.VMEM)
    sm = pl.BlockSpec(memory_space=pltpu.SMEM)
    out = pl.pallas_call(
        _fused_body,
        out_shape=jax.ShapeDtypeStruct((rows, 128), jnp.float32),
        grid=(grid,),
        in_specs=[vm, vm, vm, vm, sm, sm, sm, vm, blk],
        out_specs=blk,
        scratch_shapes=[pltpu.VMEM((8, 128), jnp.float32),
                        pltpu.VMEM((8, 128), jnp.int32)],
        compiler_params=pltpu.CompilerParams(
            dimension_semantics=("arbitrary",),
            vmem_limit_bytes=32 << 20,
        ),
    )(gender_emb.astype(f32), college_emb.astype(f32),
      school_emb.astype(f32), mbti_emb.astype(f32),
      final_fc_w.astype(f32), final_fc_b.astype(f32), weight.astype(f32),
      jnp.asarray(_MASKS), packed)

    return out.reshape(-1)[:B].reshape(B, 1)
